# baseline (speedup 1.0000x reference)
import functools

import jax
import jax.numpy as jnp
from jax.experimental import pallas as pl
from jax.experimental.pallas import tpu as pltpu

_LN_EPS = 1e-5
_N_CHUNKS = 4


def _moe_kernel(x_ref, w_ref, gw_ref, gamma_ref, beta_ref, o_ref,
                acc_sc, ca_sc, cb_sc, *, num_experts):
    i = pl.program_id(0)
    p = jax.lax.rem(i, 2)
    h = w_ref.shape[1]
    ch = h // _N_CHUNKS

    x = x_ref[...].astype(jnp.bfloat16)
    coef_a = ca_sc[1 - p]
    coef_b = cb_sc[1 - p]

    sums = []
    sqs = []
    for c in range(_N_CHUNKS):
        cols = pl.ds(c * ch, ch)
        acc_c = jnp.dot(x, w_ref[:, cols], preferred_element_type=jnp.float32)
        acc_sc[p, :, cols] = acc_c.astype(jnp.bfloat16)
        sums.append(jnp.sum(acc_c, axis=-1, keepdims=True))
        sqs.append(jnp.sum(acc_c * acc_c, axis=-1, keepdims=True))
        prev_c = acc_sc[1 - p, :, cols].astype(jnp.float32)
        o_ref[:, cols] = ((prev_c * coef_a + coef_b)
                          * gamma_ref[:, cols] + beta_ref[:, cols])

    logits = jax.lax.dot_general(
        x, gw_ref[...], (((1,), (1,)), ((), ())),
        preferred_element_type=jnp.float32)
    m1 = jnp.max(logits, axis=-1, keepdims=True)
    iota = jax.lax.broadcasted_iota(jnp.int32, logits.shape, 1)
    is_max = logits == m1
    first_idx = jnp.min(jnp.where(is_max, iota, num_experts),
                        axis=-1, keepdims=True)
    masked = jnp.where(iota == first_idx, -jnp.inf, logits)
    m2 = jnp.max(masked, axis=-1, keepdims=True)
    s = m1 + m2

    inv_h = 1.0 / h
    mu = sum(sums) * inv_h
    msq = sum(sqs) * inv_h
    var = msq - mu * mu
    ca = s * jax.lax.rsqrt(s * s * var + _LN_EPS)
    ca_sc[p] = ca
    cb_sc[p] = -mu * ca


@jax.jit
def kernel(hidden_states, gate_w, expert_weight, ln_gamma, ln_beta):
    b, s, h = hidden_states.shape
    e = gate_w.shape[0]
    m = b * s
    bm = min(256, m)
    m_blocks = m // bm

    x2d = hidden_states.reshape(m, h)
    w16 = expert_weight.astype(jnp.bfloat16)
    gw16 = gate_w.astype(jnp.bfloat16)
    gamma2d = ln_gamma.reshape(1, h)
    beta2d = ln_beta.reshape(1, h)

    last = m_blocks - 1
    out = pl.pallas_call(
        functools.partial(_moe_kernel, num_experts=e),
        grid=(m_blocks + 1,),
        in_specs=[
            pl.BlockSpec((bm, h), lambda i: (jnp.minimum(i, last), 0)),
            pl.BlockSpec((h, h), lambda i: (0, 0)),
            pl.BlockSpec((e, h), lambda i: (0, 0)),
            pl.BlockSpec((1, h), lambda i: (0, 0)),
            pl.BlockSpec((1, h), lambda i: (0, 0)),
        ],
        out_specs=pl.BlockSpec((bm, h), lambda i: (jnp.maximum(i - 1, 0), 0)),
        out_shape=jax.ShapeDtypeStruct((m, h), jnp.float32),
        scratch_shapes=[
            pltpu.VMEM((2, bm, h), jnp.bfloat16),
            pltpu.VMEM((2, bm, 1), jnp.float32),
            pltpu.VMEM((2, bm, 1), jnp.float32),
        ],
        compiler_params=pltpu.CompilerParams(
            dimension_semantics=("arbitrary",)),
    )(x2d, w16, gw16, gamma2d, beta2d)

    return out.reshape(b, s, h)
